# baseline (speedup 1.0000x reference)
import functools

import jax
import jax.numpy as jnp
from jax import lax
from jax.experimental import pallas as pl
from jax.experimental.pallas import tpu as pltpu
from jax.experimental.pallas import tpu_sc as plsc

NC = 2
NS = 16
L = 16
NW = NC * NS

B = 16384
D = 64
V = 1_000_000
BPW = B // NW
CHUNK = 128
NCHUNK = BPW // CHUNK

PACK_BC = 2048
HALF = 524288
HGRID = HALF // PACK_BC


def _pack_body(a_ref, b_ref, out_ref):
    eye = (lax.broadcasted_iota(jnp.int32, (D, D), 0)
           == lax.broadcasted_iota(jnp.int32, (D, D), 1)).astype(jnp.float32)
    dn = (((0,), (0,)), ((), ()))
    ya = lax.dot_general(a_ref[...], eye, dn,
                         preferred_element_type=jnp.float32)
    yb = lax.dot_general(b_ref[...], eye, dn,
                         preferred_element_type=jnp.float32)
    out_ref[:, 0:64] = ya
    out_ref[:, 64:128] = yb


def _pack_table(tab_t):
    return pl.pallas_call(
        _pack_body,
        grid=(HGRID,),
        in_specs=[
            pl.BlockSpec((D, PACK_BC), lambda j: (0, j)),
            pl.BlockSpec(
                (D, PACK_BC),
                lambda j: (0, jnp.minimum(j + HGRID,
                                          (V + PACK_BC - 1) // PACK_BC - 1))),
        ],
        out_specs=pl.BlockSpec((PACK_BC, 128), lambda j: (j, 0)),
        out_shape=jax.ShapeDtypeStruct((HALF, 128), jnp.float32),
        compiler_params=pltpu.CompilerParams(
            fuse_transposed_lhs_in_matmul=True),
    )(tab_t, tab_t)


def _lane_perm(v, idx):
    return lax.gather(
        v, idx[:, None],
        lax.GatherDimensionNumbers(offset_dims=(), collapsed_slice_dims=(0,),
                                   start_index_map=(0,)),
        (1,), mode=lax.GatherScatterMode.PROMISE_IN_BOUNDS)


def _gather_body(users_hbm, items_hbm, utab_hbm, itab_hbm,
                 uout_hbm, iout_hbm, r_hbm,
                 uidx_v, iidx_v, uhalf_v, ihalf_v, upair_v, ipair_v,
                 urows_v, irows_v, rhat_v, sem):
    wid = lax.axis_index("s") * NC + lax.axis_index("c")
    base = wid * BPW

    pltpu.sync_copy(users_hbm.at[pl.ds(base, BPW)], uidx_v)
    pltpu.sync_copy(items_hbm.at[pl.ds(base, BPW)], iidx_v)

    def half_body(m, carry):
        sl = pl.ds(m * L, L)
        uhalf_v[sl] = uidx_v[sl] & (HALF - 1)
        ihalf_v[sl] = iidx_v[sl] & (HALF - 1)
        return carry

    lax.fori_loop(0, BPW // L, half_body, 0)

    lane_iota = lax.iota(jnp.int32, L)
    perms = [lane_iota ^ sh for sh in (8, 4, 2, 1)]

    for c in range(NCHUNK):
        csl = pl.ds(c * CHUNK, CHUNK)
        cp_u = pltpu.async_copy(utab_hbm.at[uhalf_v.at[csl]], upair_v, sem)
        cp_i = pltpu.async_copy(itab_hbm.at[ihalf_v.at[csl]], ipair_v, sem)
        cp_u.wait()
        cp_i.wait()

        def group_body(g, carry):
            gsl = pl.ds(c * CHUNK + g * L, L)
            upar = uidx_v[gsl] & HALF
            ipar = iidx_v[gsl] & HALF
            vec = jnp.zeros((L,), jnp.float32)
            for j in range(L):
                r = g * L + j
                usel = upar[j] != 0
                isel = ipar[j] != 0
                acc = jnp.zeros((L,), jnp.float32)
                for m in range(4):
                    u16 = jnp.where(usel,
                                    upair_v[r, pl.ds(64 + m * 16, 16)],
                                    upair_v[r, pl.ds(m * 16, 16)])
                    i16 = jnp.where(isel,
                                    ipair_v[r, pl.ds(64 + m * 16, 16)],
                                    ipair_v[r, pl.ds(m * 16, 16)])
                    urows_v[r, pl.ds(m * 16, 16)] = u16
                    irows_v[r, pl.ds(m * 16, 16)] = i16
                    acc = acc + u16 * i16
                for p in perms:
                    acc = acc + _lane_perm(acc, p)
                vec = jnp.where(lane_iota == j, acc, vec)
            rhat_v[pl.ds(g * L, L)] = vec
            return carry

        lax.fori_loop(0, CHUNK // L, group_body, 0)

        obase = base + c * CHUNK
        pltpu.sync_copy(urows_v, uout_hbm.at[pl.ds(obase, CHUNK)])
        pltpu.sync_copy(irows_v, iout_hbm.at[pl.ds(obase, CHUNK)])
        pltpu.sync_copy(rhat_v, r_hbm.at[pl.ds(obase, CHUNK)])


@jax.jit
def kernel(users, items, user_embeddings, item_embeddings):
    utab = _pack_table(user_embeddings.T)
    itab = _pack_table(item_embeddings.T)
    mesh = plsc.VectorSubcoreMesh(core_axis_name="c", subcore_axis_name="s")
    f = pl.kernel(
        _gather_body,
        mesh=mesh,
        out_type=(
            jax.ShapeDtypeStruct((B, D), jnp.float32),
            jax.ShapeDtypeStruct((B, D), jnp.float32),
            jax.ShapeDtypeStruct((B,), jnp.float32),
        ),
        scratch_types=[
            pltpu.VMEM((BPW,), jnp.int32),
            pltpu.VMEM((BPW,), jnp.int32),
            pltpu.VMEM((BPW,), jnp.int32),
            pltpu.VMEM((BPW,), jnp.int32),
            pltpu.VMEM((CHUNK, 128), jnp.float32),
            pltpu.VMEM((CHUNK, 128), jnp.float32),
            pltpu.VMEM((CHUNK, D), jnp.float32),
            pltpu.VMEM((CHUNK, D), jnp.float32),
            pltpu.VMEM((CHUNK,), jnp.float32),
            pltpu.SemaphoreType.DMA,
        ],
        compiler_params=pltpu.CompilerParams(use_tc_tiling_on_sc=False),
    )
    return f(users, items, utab, itab)


# speedup vs baseline: 1.3930x; 1.3930x over previous
import functools

import jax
import jax.numpy as jnp
from jax import lax
from jax.experimental import pallas as pl
from jax.experimental.pallas import tpu as pltpu
from jax.experimental.pallas import tpu_sc as plsc

NC = 2
NS = 16
L = 16
NW = NC * NS

B = 16384
D = 64
V = 1_000_000
BPW = B // NW
CHUNK = 128
NCHUNK = BPW // CHUNK

PACK_BC = 8192
HALF = 524288
HGRID = HALF // PACK_BC


def _pack_body(a_ref, b_ref, out_ref):
    eye = (lax.broadcasted_iota(jnp.int32, (D, D), 0)
           == lax.broadcasted_iota(jnp.int32, (D, D), 1)).astype(jnp.float32)
    dn = (((0,), (0,)), ((), ()))
    ya = lax.dot_general(a_ref[...], eye, dn,
                         preferred_element_type=jnp.float32)
    yb = lax.dot_general(b_ref[...], eye, dn,
                         preferred_element_type=jnp.float32)
    out_ref[:, 0:64] = ya
    out_ref[:, 64:128] = yb


def _pack_table(tab_t):
    return pl.pallas_call(
        _pack_body,
        grid=(HGRID,),
        in_specs=[
            pl.BlockSpec((D, PACK_BC), lambda j: (0, j)),
            pl.BlockSpec(
                (D, PACK_BC),
                lambda j: (0, jnp.minimum(j + HGRID,
                                          (V + PACK_BC - 1) // PACK_BC - 1))),
        ],
        out_specs=pl.BlockSpec((PACK_BC, 128), lambda j: (j, 0)),
        out_shape=jax.ShapeDtypeStruct((HALF, 128), jnp.float32),
        compiler_params=pltpu.CompilerParams(
            fuse_transposed_lhs_in_matmul=True),
    )(tab_t, tab_t)


def _lane_perm(v, idx):
    return lax.gather(
        v, idx[:, None],
        lax.GatherDimensionNumbers(offset_dims=(), collapsed_slice_dims=(0,),
                                   start_index_map=(0,)),
        (1,), mode=lax.GatherScatterMode.PROMISE_IN_BOUNDS)


def _gather_body(users_hbm, items_hbm, utab_hbm, itab_hbm,
                 uout_hbm, iout_hbm, r_hbm,
                 uidx_v, iidx_v, uhalf_v, ihalf_v,
                 upair_v, urows_v, ipair_v, irows_v,
                 rhat_v, sem):
    wid = lax.axis_index("s") * NC + lax.axis_index("c")
    base = wid * BPW

    pltpu.sync_copy(users_hbm.at[pl.ds(base, BPW)], uidx_v)
    pltpu.sync_copy(items_hbm.at[pl.ds(base, BPW)], iidx_v)

    def half_body(m, carry):
        sl = pl.ds(m * L, L)
        uhalf_v[sl] = uidx_v[sl] & (HALF - 1)
        ihalf_v[sl] = iidx_v[sl] & (HALF - 1)
        return carry

    lax.fori_loop(0, BPW // L, half_body, 0)

    lane_iota = lax.iota(jnp.int32, L)
    perms = [lane_iota ^ sh for sh in (8, 4, 2, 1)]

    for c in range(NCHUNK):
        csl = pl.ds(c * CHUNK, CHUNK)
        cp_u = pltpu.async_copy(utab_hbm.at[uhalf_v.at[csl]], upair_v, sem)
        cp_i = pltpu.async_copy(itab_hbm.at[ihalf_v.at[csl]], ipair_v, sem)
        cp_u.wait()
        cp_i.wait()

        def group_body(g, carry):
            gsl = pl.ds(c * CHUNK + g * L, L)
            upar = uidx_v[gsl] & HALF
            ipar = iidx_v[gsl] & HALF
            vec = jnp.zeros((L,), jnp.float32)
            for j in range(L):
                r = g * L + j
                usel = upar[j] != 0
                isel = ipar[j] != 0
                acc = jnp.zeros((L,), jnp.float32)
                for m in range(4):
                    u16 = jnp.where(usel,
                                    upair_v[r, pl.ds(64 + m * 16, 16)],
                                    upair_v[r, pl.ds(m * 16, 16)])
                    i16 = jnp.where(isel,
                                    ipair_v[r, pl.ds(64 + m * 16, 16)],
                                    ipair_v[r, pl.ds(m * 16, 16)])
                    urows_v[r, pl.ds(m * 16, 16)] = u16
                    irows_v[r, pl.ds(m * 16, 16)] = i16
                    acc = acc + u16 * i16
                for p in perms:
                    acc = acc + _lane_perm(acc, p)
                vec = jnp.where(lane_iota == j, acc, vec)
            rhat_v[pl.ds(g * L, L)] = vec
            return carry

        lax.fori_loop(0, CHUNK // L, group_body, 0)

        obase = base + c * CHUNK
        pltpu.sync_copy(urows_v, uout_hbm.at[pl.ds(obase, CHUNK)])
        pltpu.sync_copy(irows_v, iout_hbm.at[pl.ds(obase, CHUNK)])
        pltpu.sync_copy(rhat_v, r_hbm.at[pl.ds(obase, CHUNK)])


@jax.jit
def kernel(users, items, user_embeddings, item_embeddings):
    utab = _pack_table(user_embeddings.T)
    itab = _pack_table(item_embeddings.T)
    mesh = plsc.VectorSubcoreMesh(core_axis_name="c", subcore_axis_name="s")
    f = pl.kernel(
        _gather_body,
        mesh=mesh,
        out_type=(
            jax.ShapeDtypeStruct((B, D), jnp.float32),
            jax.ShapeDtypeStruct((B, D), jnp.float32),
            jax.ShapeDtypeStruct((B,), jnp.float32),
        ),
        scratch_types=[
            pltpu.VMEM((BPW,), jnp.int32),
            pltpu.VMEM((BPW,), jnp.int32),
            pltpu.VMEM((BPW,), jnp.int32),
            pltpu.VMEM((BPW,), jnp.int32),
            pltpu.VMEM((CHUNK, 128), jnp.float32),
            pltpu.VMEM((CHUNK, D), jnp.float32),
            pltpu.VMEM((CHUNK, 128), jnp.float32),
            pltpu.VMEM((CHUNK, D), jnp.float32),
            pltpu.VMEM((CHUNK,), jnp.float32),
            pltpu.SemaphoreType.DMA,
        ],
        compiler_params=pltpu.CompilerParams(use_tc_tiling_on_sc=False),
    )
    return f(users, items, utab, itab)
